# baseline (speedup 1.0000x reference)
import jax
import jax.numpy as jnp
from jax.experimental import pallas as pl
from jax.experimental.pallas import tpu as pltpu

D_MODEL = 2048
N_EXP = 64
TOPK = 8
N_TOK = 8192
BLK = 512


def _router_body(x_ref, w_ref, logits_ref, scores_ref, ew_ref, ei_ref, hist_ref):
    x = x_ref[...]
    w = w_ref[...]
    logits = jax.lax.dot_general(
        x, w, (((1,), (1,)), ((), ())), preferred_element_type=jnp.float32
    )
    logits_ref[...] = logits

    m = jnp.max(logits, axis=-1, keepdims=True)
    e = jnp.exp(logits - m)
    s = e / jnp.sum(e, axis=-1, keepdims=True)
    scores_ref[...] = s

    iota_f = jax.lax.broadcasted_iota(jnp.int32, (BLK, N_EXP), 1).astype(jnp.float32)
    work = s
    ew_cols = []
    ei_cols = []
    for _ in range(TOPK):
        mx = jnp.max(work, axis=-1, keepdims=True)
        eq = work == mx
        idxf = jnp.min(jnp.where(eq, iota_f, 128.0), axis=-1, keepdims=True)
        ew_cols.append(mx)
        ei_cols.append(idxf.astype(jnp.int32))
        work = jnp.where(eq, -1.0, work)
    ew_ref[...] = jnp.concatenate(ew_cols, axis=1)
    ei_ref[...] = jnp.concatenate(ei_cols, axis=1)
    hist = jnp.sum((work == -1.0).astype(jnp.int32), axis=0, keepdims=True)

    @pl.when(pl.program_id(0) == 0)
    def _():
        hist_ref[...] = jnp.zeros_like(hist_ref)

    hist_ref[...] += hist


def kernel(x, W):
    grid = (N_TOK // BLK,)
    logits, scores, ew, ei, hist = pl.pallas_call(
        _router_body,
        grid=grid,
        in_specs=[
            pl.BlockSpec((BLK, D_MODEL), lambda i: (i, 0)),
            pl.BlockSpec((N_EXP, D_MODEL), lambda i: (0, 0)),
        ],
        out_specs=[
            pl.BlockSpec((BLK, N_EXP), lambda i: (i, 0)),
            pl.BlockSpec((BLK, N_EXP), lambda i: (i, 0)),
            pl.BlockSpec((BLK, TOPK), lambda i: (i, 0)),
            pl.BlockSpec((BLK, TOPK), lambda i: (i, 0)),
            pl.BlockSpec((1, N_EXP), lambda i: (0, 0)),
        ],
        out_shape=[
            jax.ShapeDtypeStruct((N_TOK, N_EXP), jnp.float32),
            jax.ShapeDtypeStruct((N_TOK, N_EXP), jnp.float32),
            jax.ShapeDtypeStruct((N_TOK, TOPK), jnp.float32),
            jax.ShapeDtypeStruct((N_TOK, TOPK), jnp.int32),
            jax.ShapeDtypeStruct((1, N_EXP), jnp.int32),
        ],
    )(x, W)
    return logits, scores, ew, ei, hist.reshape(N_EXP)


# speedup vs baseline: 1.0954x; 1.0954x over previous
import jax
import jax.numpy as jnp
from jax.experimental import pallas as pl
from jax.experimental.pallas import tpu as pltpu

D_MODEL = 2048
N_EXP = 64
TOPK = 8
N_TOK = 8192
BLK = 1024


def _router_body(x_ref, w_ref, logits_ref, scores_ref, ew_ref, ei_ref, hist_ref):
    x = x_ref[...]
    w = w_ref[...]
    logits = jax.lax.dot_general(
        x, w, (((1,), (1,)), ((), ())), preferred_element_type=jnp.float32
    )
    logits_ref[...] = logits

    m = jnp.max(logits, axis=-1, keepdims=True)
    e = jnp.exp(logits - m)
    s = e / jnp.sum(e, axis=-1, keepdims=True)
    scores_ref[...] = s

    iota_f = jax.lax.broadcasted_iota(jnp.int32, (BLK, N_EXP), 1).astype(jnp.float32)
    work = s
    ew_cols = []
    ei_cols = []
    for _ in range(TOPK):
        mx = jnp.max(work, axis=-1, keepdims=True)
        eq = work == mx
        idxf = jnp.min(jnp.where(eq, iota_f, 128.0), axis=-1, keepdims=True)
        ew_cols.append(mx)
        ei_cols.append(idxf.astype(jnp.int32))
        work = jnp.where(eq, -1.0, work)
    ew_ref[...] = jnp.concatenate(ew_cols, axis=1)
    ei_ref[...] = jnp.concatenate(ei_cols, axis=1)
    hist = jnp.sum((work == -1.0).astype(jnp.int32), axis=0, keepdims=True)

    @pl.when(pl.program_id(0) == 0)
    def _():
        hist_ref[...] = jnp.zeros_like(hist_ref)

    hist_ref[...] += hist


def kernel(x, W):
    grid = (N_TOK // BLK,)
    logits, scores, ew, ei, hist = pl.pallas_call(
        _router_body,
        grid=grid,
        in_specs=[
            pl.BlockSpec((BLK, D_MODEL), lambda i: (i, 0)),
            pl.BlockSpec((N_EXP, D_MODEL), lambda i: (0, 0)),
        ],
        out_specs=[
            pl.BlockSpec((BLK, N_EXP), lambda i: (i, 0)),
            pl.BlockSpec((BLK, N_EXP), lambda i: (i, 0)),
            pl.BlockSpec((BLK, TOPK), lambda i: (i, 0)),
            pl.BlockSpec((BLK, TOPK), lambda i: (i, 0)),
            pl.BlockSpec((1, N_EXP), lambda i: (0, 0)),
        ],
        out_shape=[
            jax.ShapeDtypeStruct((N_TOK, N_EXP), jnp.float32),
            jax.ShapeDtypeStruct((N_TOK, N_EXP), jnp.float32),
            jax.ShapeDtypeStruct((N_TOK, TOPK), jnp.float32),
            jax.ShapeDtypeStruct((N_TOK, TOPK), jnp.int32),
            jax.ShapeDtypeStruct((1, N_EXP), jnp.int32),
        ],
    )(x, W)
    return logits, scores, ew, ei, hist.reshape(N_EXP)
